# baseline (speedup 1.0000x reference)
import jax
import jax.numpy as jnp
from jax.experimental import pallas as pl

_NUM_SEGMENTS = 100000
_EPS = 1e-08


def kernel(multi_dim_pressures, weights, group_ids, running_mean, running_var):
    x = multi_dim_pressures
    N, K = x.shape
    ones = jnp.ones((N,), dtype=x.dtype)
    counts = jax.ops.segment_sum(ones, group_ids, num_segments=_NUM_SEGMENTS)
    safe_counts = jnp.maximum(counts, 1.0)
    seg_sum = jax.ops.segment_sum(x, group_ids, num_segments=_NUM_SEGMENTS)
    mean = seg_sum / safe_counts[:, None]
    mean_g = jnp.take(mean, group_ids, axis=0)
    diff = x - mean_g
    seg_sq = jax.ops.segment_sum(diff * diff, group_ids, num_segments=_NUM_SEGMENTS)
    var = seg_sq / safe_counts[:, None]
    std = jnp.sqrt(jnp.maximum(var, 0.0)) + _EPS
    std_g = jnp.take(std, group_ids, axis=0)
    cnt_g = jnp.take(counts, group_ids, axis=0)
    group_normed = diff / std_g
    fallback = (x - running_mean[None, :]) / (jnp.sqrt(running_var)[None, :] + _EPS)
    decoupled = jnp.where((cnt_g > 1.0)[:, None], group_normed, fallback)
    return decoupled


# speedup vs baseline: 12.4201x; 12.4201x over previous
import functools

import jax
import jax.numpy as jnp
from jax import lax
from jax.experimental import pallas as pl
from jax.experimental.pallas import tpu as pltpu
from jax.experimental.pallas import tpu_sc as plsc

_SEG = 100000
_EPS = 1e-08
_NC = 2
_NS = 16
_NW = _NC * _NS
_SQRT_MAGIC = 0x1FBD1DF5


def _dg(v, idx):
    return lax.gather(
        v,
        idx[:, None],
        lax.GatherDimensionNumbers(
            offset_dims=(), collapsed_slice_dims=(0,), start_index_map=(0,)
        ),
        (1,),
        mode=lax.GatherScatterMode.PROMISE_IN_BOUNDS,
    )


def _lane_iota():
    return lax.iota(jnp.int32, 16)


def _bcast_lane(v, lane):
    return _dg(v, jnp.full((16,), lane, dtype=jnp.int32))


def _inv_std_from_var(var):
    bits = lax.bitcast_convert_type(var, jnp.int32)
    s = lax.bitcast_convert_type((bits >> 1) + _SQRT_MAGIC, jnp.float32)
    s = 0.5 * (s + var / s)
    s = 0.5 * (s + var / s)
    s = 0.5 * (s + var / s)
    return 1.0 / (s + _EPS)


def _build_k1(n_rows, n_seg, tile, ):
    chunk = n_rows // _NW
    assert chunk % tile == 0 and tile % 16 == 0
    groups = tile // 16
    mesh = plsc.VectorSubcoreMesh(
        core_axis_name="c", subcore_axis_name="s",
        num_cores=_NC, num_subcores=_NS,
    )

    def body(x_hbm, ids16_hbm, fbm_hbm, fbi_hbm, params_hbm,
             xt, idt, fpb, fib, fib2, pga, pgb, fbmv, fbiv):
        w = lax.axis_index("c") * _NS + lax.axis_index("s")
        base = w * chunk
        stop_row = base + chunk - 1
        lane = _lane_iota()
        shift_idx = jnp.minimum(lane + 1, 15)
        pltpu.sync_copy(fbm_hbm, fbmv)
        pltpu.sync_copy(fbi_hbm, fbiv)
        fbm = fbmv[:]
        fbi = fbiv[:]

        pltpu.sync_copy(
            ids16_hbm.at[pl.ds(pl.multiple_of(jnp.maximum(base - 16, 0), 16),
                               16)], pga)
        pltpu.sync_copy(ids16_hbm.at[pl.ds(pl.multiple_of(base, 16), 16)], pgb)
        prev_last = _bcast_lane(pga[:], jnp.int32(15))
        cur_first = _bcast_lane(pgb[:], jnp.int32(0))
        neq = jnp.max((prev_last != cur_first).astype(jnp.int32))
        emit0 = jnp.where(w == 0, jnp.int32(1), neq)

        for i in range(8):
            fib2[i, :] = jnp.zeros((16,), jnp.int32)

        zeros = jnp.zeros((16,), jnp.float32)

        def flush_all():
            for i in range(8):
                fib[pl.ds(i * 16, 16)] = fib2[i, :]
            pltpu.sync_copy(fpb, params_hbm.at[fib])
            for i in range(8):
                fib2[i, :] = jnp.zeros((16,), jnp.int32)

        def o_cond(st):
            return (st[6] == 0) & (st[0] < n_rows)

        def o_body(st):
            t0, cnt, sm, sq, emit, cursor, done = st
            t0a = pl.multiple_of(t0, tile)
            pltpu.sync_copy(x_hbm.at[pl.ds(t0a, tile)], xt)
            pltpu.sync_copy(ids16_hbm.at[pl.ds(t0a, tile + 16)], idt)

            def group_fn(l, gst):
                cnt, sm, sq, emit, cursor, done = gst
                v = idt[pl.ds(pl.multiple_of(l * 16, 16), 16)]
                vn = idt[pl.ds(pl.multiple_of(l * 16 + 16, 16), 16)]
                shifted = _dg(v, shift_idx)
                nxt = jnp.where(lane == 15, _bcast_lane(vn, jnp.int32(0)),
                                shifted)
                endm = v != nxt

                def q_cond(qs):
                    return (qs[0] < 16) & (qs[6] == 0)

                def q_body(qs):
                    q, cnt, sm, sq, emit, cursor, done = qs
                    sel = endm & (lane >= q)
                    ne = jnp.max(plsc.all_reduce_ffs(sel))
                    limit = jnp.minimum(ne + 1, 16)
                    rbase = l * 16

                    def acc(j, c):
                        s_, q_ = c
                        row = xt[rbase + j, :]
                        return (s_ + row, q_ + row * row)

                    sm, sq = lax.fori_loop(q, limit, acc, (sm, sq))
                    cnt = cnt + jnp.full((16,),
                                         (limit - q).astype(jnp.float32))
                    fin = ne < 16
                    safec = jnp.maximum(cnt, 1.0)
                    mean = sm / safec
                    var = jnp.maximum(sq / safec - mean * mean, 0.0)
                    inv = _inv_std_from_var(var)
                    grp = cnt > 1.5
                    pm = jnp.where(grp, mean, fbm)
                    pv = jnp.where(grp, inv, fbi)
                    rid = _bcast_lane(v, jnp.minimum(ne, 15))
                    do_emit = fin & (emit == 1) & (done == 0)

                    @pl.when(do_emit)
                    def _():
                        fpb[cursor, 0:16] = pm
                        fpb[cursor, 16:32] = pv
                        cr = cursor // 16
                        cl = cursor % 16
                        fib2[cr, :] = (fib2[cr, :]
                                       + rid * (lane == cl).astype(jnp.int32))

                    ncur = jnp.where(do_emit, cursor + 1, cursor)

                    @pl.when(ncur == 128)
                    def _():
                        flush_all()

                    cursor = jnp.where(ncur == 128, jnp.int32(0), ncur)
                    rend = t0 + rbase + ne
                    done = jnp.where(fin & (rend >= stop_row),
                                     jnp.int32(1), done)
                    emit = jnp.where(fin, jnp.int32(1), emit)
                    cnt = jnp.where(fin, zeros, cnt)
                    sm = jnp.where(fin, zeros, sm)
                    sq = jnp.where(fin, zeros, sq)
                    return (limit, cnt, sm, sq, emit, cursor, done)

                out = lax.while_loop(
                    q_cond, q_body,
                    (jnp.int32(0), cnt, sm, sq, emit, cursor, done))
                return out[1:]

            cnt, sm, sq, emit, cursor, done = lax.fori_loop(
                0, groups, group_fn, (cnt, sm, sq, emit, cursor, done))
            return (t0 + tile, cnt, sm, sq, emit, cursor, done)

        st = lax.while_loop(
            o_cond, o_body,
            (base, zeros, zeros, zeros, emit0, jnp.int32(0), jnp.int32(0)))
        cursor = st[5]

        @pl.when(cursor > 0)
        def _():
            last = cursor - 1
            lm = fpb[last, 0:16]
            li = fpb[last, 16:32]
            lid = _bcast_lane(fib2[last // 16, :], last % 16)

            def padj(j, carry):
                fpb[j, 0:16] = lm
                fpb[j, 16:32] = li
                fib2[j // 16, :] = (fib2[j // 16, :]
                                    + lid * (lane == (j % 16)).astype(jnp.int32))
                return carry

            lax.fori_loop(cursor, 128, padj, jnp.int32(0))
            flush_all()

    return pl.kernel(
        body,
        out_type=jax.ShapeDtypeStruct((n_seg, 32), jnp.float32),
        mesh=mesh,
        compiler_params=pltpu.CompilerParams(use_tc_tiling_on_sc=False, needs_layout_passes=False),
        scratch_types=[
            pltpu.VMEM((tile, 16), jnp.float32),
            pltpu.VMEM((tile + 16,), jnp.int32),
            pltpu.VMEM((128, 32), jnp.float32),
            pltpu.VMEM((128,), jnp.int32),
            pltpu.VMEM((8, 16), jnp.int32),
            pltpu.VMEM((16,), jnp.int32),
            pltpu.VMEM((16,), jnp.int32),
            pltpu.VMEM((16,), jnp.float32),
            pltpu.VMEM((16,), jnp.float32),
        ],
    )


def _build_k2(n_rows, n_seg, tile):
    chunk = n_rows // _NW
    assert chunk % tile == 0 and tile % 125 == 0
    nbatch = tile // 125
    ntiles = chunk // tile
    mesh = plsc.VectorSubcoreMesh(
        core_axis_name="c", subcore_axis_name="s",
        num_cores=_NC, num_subcores=_NS,
    )

    def body(x_hbm, ids125_hbm, params_hbm, out_hbm, xt, ot, idt, pt, sem):
        w = lax.axis_index("c") * _NS + lax.axis_index("s")
        base = w * chunk

        def tile_fn(k, carry):
            t0 = pl.multiple_of(base + k * tile, tile)
            pltpu.sync_copy(x_hbm.at[pl.ds(t0, tile)], xt)
            pltpu.sync_copy(
                ids125_hbm.at[pl.ds(pl.multiple_of(t0 // 125, nbatch), nbatch)],
                idt)
            copies = [
                pltpu.make_async_copy(
                    params_hbm.at[idt.at[b]],
                    pt.at[pl.ds(b * 125, 125)],
                    sem,
                )
                for b in range(nbatch)
            ]
            for c in copies:
                c.start()
            for c in copies:
                c.wait()

            def rbody(r, cc):
                ot[r, :] = (xt[r, :] - pt[r, 0:16]) * pt[r, 16:32]
                return cc

            lax.fori_loop(0, tile, rbody, jnp.int32(0))
            pltpu.sync_copy(ot, out_hbm.at[pl.ds(t0, tile)])
            return carry

        lax.fori_loop(0, ntiles, tile_fn, jnp.int32(0))

    return pl.kernel(
        body,
        out_type=jax.ShapeDtypeStruct((n_rows, 16), jnp.float32),
        mesh=mesh,
        compiler_params=pltpu.CompilerParams(use_tc_tiling_on_sc=False, needs_layout_passes=False),
        scratch_types=[
            pltpu.VMEM((tile, 16), jnp.float32),
            pltpu.VMEM((tile, 16), jnp.float32),
            pltpu.VMEM((nbatch, 125), jnp.int32),
            pltpu.VMEM((tile, 32), jnp.float32),
            pltpu.SemaphoreType.DMA,
        ],
    )


@functools.partial(jax.jit, static_argnames=())
def _run(x, gid, fbm, fbi):
    n_rows = x.shape[0]
    ids16 = jnp.concatenate([gid, jnp.full((16,), -1, jnp.int32)])
    ids125 = gid.reshape(n_rows // 125, 125)
    k1 = _build_k1(n_rows, _SEG, 2000)
    params = k1(x, ids16, fbm, fbi)
    k2 = _build_k2(n_rows, _SEG, 2000)
    return k2(x, ids125, params)


def kernel(multi_dim_pressures, weights, group_ids, running_mean, running_var):
    x = multi_dim_pressures
    gid = group_ids.astype(jnp.int32)
    fbm = running_mean.astype(jnp.float32)
    fbi = 1.0 / (jnp.sqrt(running_var.astype(jnp.float32)) + _EPS)
    return _run(x, gid, fbm, fbi)
